# baseline (speedup 1.0000x reference)
import functools

import jax
import jax.numpy as jnp
from jax import lax
from jax.experimental import pallas as pl
from jax.experimental.pallas import tpu as pltpu
from jax.experimental.pallas import tpu_sc as plsc

_B = 16384
_F = 26
_V = 1000000
_D = 13

_NC = 2
_NS = 16
_NW = _NC * _NS
_BPW = _B // _NW
_CH = 128
_NCH = _F * _BPW // _CH
_EPW = _F * _BPW
_GPW = _BPW // 16


def _sc_embed_linear(table_words, idx_blocks, dense_flat, wb):
    mesh = plsc.VectorSubcoreMesh(core_axis_name="c", subcore_axis_name="s")

    @functools.partial(
        pl.kernel,
        mesh=mesh,
        compiler_params=pltpu.CompilerParams(needs_layout_passes=False),
        out_type=jax.ShapeDtypeStruct((_B,), jnp.float32),
        scratch_types=[
            pltpu.VMEM((_EPW,), jnp.int32),
            pltpu.VMEM((_EPW,), jnp.int32),
            pltpu.VMEM((_EPW,), jnp.int32),
            pltpu.VMEM((_D * _BPW,), jnp.float32),
            pltpu.VMEM((_F + _D + 1, 16), jnp.float32),
            pltpu.VMEM((_BPW,), jnp.float32),
            pltpu.SemaphoreType.DMA,
        ],
    )
    def k(table_hbm, idx_hbm, dense_hbm, wb_hbm, out_hbm,
          idx_v, widx_v, g_v, dense_v, wb_v, out_v, sem):
        wid = lax.axis_index("s") * _NC + lax.axis_index("c")

        pltpu.sync_copy(idx_hbm.at[pl.ds(wid * _EPW, _EPW)], idx_v)

        def fire(j, carry):
            for s in range(_CH // 16):
                o = j * _CH + s * 16
                widx_v[pl.ds(o, 16)] = lax.shift_right_logical(
                    idx_v[pl.ds(o, 16)], 1)
            pltpu.make_async_copy(
                table_hbm.at[widx_v.at[pl.ds(j * _CH, _CH)]],
                g_v.at[pl.ds(j * _CH, _CH)], sem).start()
            return carry
        lax.fori_loop(0, _NCH, fire, 0)

        pltpu.sync_copy(dense_hbm.at[pl.ds(wid * _D * _BPW, _D * _BPW)],
                        dense_v)
        pltpu.sync_copy(wb_hbm, wb_v)

        iota = lax.iota(jnp.int32, 16)
        iota_f = iota * _F
        iota_d = iota * _D

        def dense_part(s, carry):
            acc = wb_v[_F + _D]
            dbase = s * (16 * _D)
            for d in range(_D):
                v = plsc.load_gather(dense_v, [iota_d + (dbase + d)])
                acc = acc + v * wb_v[_F + d]
            out_v[pl.ds(s * 16, 16)] = acc
            return carry
        lax.fori_loop(0, _GPW, dense_part, 0)

        def drain(j, carry):
            pltpu.make_async_copy(
                table_hbm.at[widx_v.at[pl.ds(j * _CH, _CH)]],
                g_v.at[pl.ds(j * _CH, _CH)], sem).wait()
            return carry
        lax.fori_loop(0, _NCH, drain, 0)

        himask = jnp.full((16,), -65536, jnp.int32)
        sixteen = jnp.full((16,), 16, jnp.int32)
        one = jnp.full((16,), 1, jnp.int32)

        def emb_part(s, carry):
            acc = out_v[pl.ds(s * 16, 16)]
            gbase = s * (16 * _F)
            for f in range(_F):
                pos = iota_f + (gbase + f)
                w_u = plsc.load_gather(g_v, [pos])
                iv = plsc.load_gather(idx_v, [pos])
                sh = (one - (iv & one)) * sixteen
                bits = lax.shift_left(w_u, sh) & himask
                val = plsc.bitcast(bits, jnp.float32)
                acc = acc + val * wb_v[f]
            out_v[pl.ds(s * 16, 16)] = acc
            return carry
        lax.fori_loop(0, _GPW, emb_part, 0)

        pltpu.sync_copy(out_v, out_hbm.at[pl.ds(wid * _BPW, _BPW)])

    return k(table_words, idx_blocks, dense_flat, wb)


def kernel(sparse_idx, dense, emb_tables, fc_w, fc_b):
    table_bf = emb_tables.astype(jnp.bfloat16).reshape(_F * _V // 2, 2)
    table_words = lax.bitcast_convert_type(table_bf, jnp.int32)
    flat_idx = (sparse_idx + jnp.arange(_F, dtype=jnp.int32)[None, :] * _V
                ).reshape(_B * _F)
    dense_flat = dense.reshape(_B * _D)
    wb = jnp.broadcast_to(
        jnp.concatenate([fc_w.reshape(-1), fc_b]).reshape(_F + _D + 1, 1),
        (_F + _D + 1, 16))
    out = _sc_embed_linear(table_words, flat_idx, dense_flat, wb)
    return out.reshape(_B, 1)


# speedup vs baseline: 11.0169x; 11.0169x over previous
import functools

import jax
import jax.numpy as jnp
from jax import lax
from jax.experimental import pallas as pl
from jax.experimental.pallas import tpu as pltpu
from jax.experimental.pallas import tpu_sc as plsc

_B = 16384
_F = 26
_V = 1000000
_D = 13

_NC = 2
_NS = 16
_NW = _NC * _NS
_BPW = _B // _NW
_CH = 128
_NCH = _F * _BPW // _CH
_EPW = _F * _BPW
_GPW = _BPW // 16


def _sc_embed_linear(table_flat, idx_blocks, dense_flat, wb):
    mesh = plsc.VectorSubcoreMesh(core_axis_name="c", subcore_axis_name="s")

    @functools.partial(
        pl.kernel,
        mesh=mesh,
        compiler_params=pltpu.CompilerParams(needs_layout_passes=False),
        out_type=jax.ShapeDtypeStruct((_B,), jnp.float32),
        scratch_types=[
            pltpu.VMEM((_EPW,), jnp.int32),
            pltpu.VMEM((_EPW,), jnp.float32),
            pltpu.VMEM((_D * _BPW,), jnp.float32),
            pltpu.VMEM((_F + _D + 1, 16), jnp.float32),
            pltpu.VMEM((_BPW,), jnp.float32),
            pltpu.SemaphoreType.DMA,
        ],
    )
    def k(table_hbm, idx_hbm, dense_hbm, wb_hbm, out_hbm,
          idx_v, g_v, dense_v, wb_v, out_v, sem):
        wid = lax.axis_index("s") * _NC + lax.axis_index("c")

        pltpu.sync_copy(idx_hbm.at[pl.ds(wid * _EPW, _EPW)], idx_v)

        def fire(j, carry):
            pltpu.make_async_copy(
                table_hbm.at[idx_v.at[pl.ds(j * _CH, _CH)]],
                g_v.at[pl.ds(j * _CH, _CH)], sem).start()
            return carry
        lax.fori_loop(0, _NCH, fire, 0)

        pltpu.sync_copy(dense_hbm.at[pl.ds(wid * _D * _BPW, _D * _BPW)],
                        dense_v)
        pltpu.sync_copy(wb_hbm, wb_v)

        iota = lax.iota(jnp.int32, 16)
        iota_f = iota * _F
        iota_d = iota * _D

        def dense_part(s, carry):
            acc = wb_v[_F + _D]
            dbase = s * (16 * _D)
            for d in range(_D):
                v = plsc.load_gather(dense_v, [iota_d + (dbase + d)])
                acc = acc + v * wb_v[_F + d]
            out_v[pl.ds(s * 16, 16)] = acc
            return carry
        lax.fori_loop(0, _GPW, dense_part, 0)

        def drain(j, carry):
            pltpu.make_async_copy(
                table_hbm.at[idx_v.at[pl.ds(j * _CH, _CH)]],
                g_v.at[pl.ds(j * _CH, _CH)], sem).wait()
            return carry
        lax.fori_loop(0, _NCH, drain, 0)

        def emb_part(s, carry):
            acc = out_v[pl.ds(s * 16, 16)]
            gbase = s * (16 * _F)
            for f in range(_F):
                v = plsc.load_gather(g_v, [iota_f + (gbase + f)])
                acc = acc + v * wb_v[f]
            out_v[pl.ds(s * 16, 16)] = acc
            return carry
        lax.fori_loop(0, _GPW, emb_part, 0)

        pltpu.sync_copy(out_v, out_hbm.at[pl.ds(wid * _BPW, _BPW)])

    return k(table_flat, idx_blocks, dense_flat, wb)


def kernel(sparse_idx, dense, emb_tables, fc_w, fc_b):
    table_flat = jnp.concatenate(
        [lax.squeeze(lax.slice_in_dim(emb_tables, f, f + 1, axis=0), (0, 2))
         for f in range(_F)])
    flat_idx = (sparse_idx + jnp.arange(_F, dtype=jnp.int32)[None, :] * _V
                ).reshape(_B * _F)
    dense_flat = dense.reshape(_B * _D)
    wb = jnp.broadcast_to(
        jnp.concatenate([fc_w.reshape(-1), fc_b]).reshape(_F + _D + 1, 1),
        (_F + _D + 1, 16))
    out = _sc_embed_linear(table_flat, flat_idx, dense_flat, wb)
    return out.reshape(_B, 1)
